# baseline (speedup 1.0000x reference)
import functools

import jax
import jax.numpy as jnp
from jax.experimental import pallas as pl
from jax.experimental.pallas import tpu as pltpu

NEG_SLOPE = 0.01
H = 32
F_UP = 128
G = 64
NK1 = 128
NKC = 512


def _leaky(x):
    return jnp.where(x > 0, x, NEG_SLOPE * x)


def _pass1_kernel(b_ref, b2_ref, xs_ref, part_ref, bb_ref, acc, *,
                  ksteps_core, nk, n):
    c = pl.program_id(0)
    k = pl.program_id(1)

    @pl.when(k == 0)
    def _():
        acc[...] = jnp.zeros_like(acc)

    bb = (b_ref[...] + b2_ref[...]).reshape(nk, n).astype(jnp.bfloat16)
    bb_ref[...] = bb
    kg = c * ksteps_core + k
    lhs = xs_ref[:, pl.ds(pl.multiple_of(kg * nk, nk), nk)]
    acc[...] += jnp.dot(lhs, bb, preferred_element_type=jnp.float32)

    @pl.when(k == ksteps_core - 1)
    def _():
        part_ref[0, :, :] = acc[...]


def _combine1_kernel(part_ref, xrow_ref, w1r_ref, b1_ref, h1_ref, dinv_ref):
    acc = jnp.sum(part_ref[...], axis=0)
    agg1 = acc[0:H, :]
    deg = acc[H:H + 1, :]
    dinv_ref[...] = jnp.where(deg > 0, 1.0 / deg, 0.0)
    h = agg1 + (w1r_ref[...] * xrow_ref[...]) + b1_ref[...]
    h1_ref[...] = _leaky(h).astype(jnp.bfloat16)


def _conv2_kernel(b_ref, ht_ref, wl_ref, wr_ref, bias_ref,
                  out_ref, acc, *, nk_steps, nk, nj):
    j = pl.program_id(0)
    k = pl.program_id(1)

    @pl.when(k == 0)
    def _():
        acc[...] = jnp.zeros_like(acc)

    lhs = ht_ref[:, pl.ds(pl.multiple_of(k * nk, nk), nk)]
    acc[...] += jnp.dot(lhs, b_ref[...], preferred_element_type=jnp.float32)

    @pl.when(k == nk_steps - 1)
    def _():
        root = ht_ref[:, pl.ds(pl.multiple_of(j * nj, nj), nj)]
        y = (jnp.dot(wl_ref[...], acc[...].astype(jnp.bfloat16),
                     preferred_element_type=jnp.float32)
             + jnp.dot(wr_ref[...], root,
                       preferred_element_type=jnp.float32)
             + bias_ref[...])
        out_ref[...] = _leaky(y).astype(jnp.bfloat16)


def _conv3_pool_kernel(b_ref, ht_ref, dinv_ref, batch_ref, pool_ref,
                       wl_ref, wr_ref, bias_ref, wu_ref, bu_ref,
                       padd_ref, pmax_ref, acc, *, nk_steps, nk, nj):
    j = pl.program_id(0)
    k = pl.program_id(1)

    @pl.when(k == 0)
    def _():
        acc[...] = jnp.zeros_like(acc)

    lhs = ht_ref[:, pl.ds(pl.multiple_of(k * nk, nk), nk)]
    acc[...] += jnp.dot(lhs, b_ref[...], preferred_element_type=jnp.float32)

    @pl.when(k == nk_steps - 1)
    def _():
        aggm = acc[...] * dinv_ref[...]
        root = ht_ref[:, pl.ds(pl.multiple_of(j * nj, nj), nj)]
        y = (jnp.dot(wl_ref[...], aggm.astype(jnp.bfloat16),
                     preferred_element_type=jnp.float32)
             + jnp.dot(wr_ref[...], root,
                       preferred_element_type=jnp.float32)
             + bias_ref[...])
        y = _leaky(y)
        z = jnp.dot(wu_ref[...], y.astype(jnp.bfloat16),
                    preferred_element_type=jnp.float32) + bu_ref[...]
        z = _leaky(z)
        zb = z.astype(jnp.bfloat16)
        padd_ref[0, :, :] = jnp.dot(zb, pool_ref[...],
                              preferred_element_type=jnp.float32)
        brow = batch_ref[...]
        neg = jnp.bfloat16(-jnp.inf)
        maxes = []
        for g in range(G):
            masked = jnp.where(brow == jnp.float32(g), zb, neg)
            maxes.append(jnp.max(masked, axis=1))
        pmax_ref[0, :, :] = jnp.stack(maxes, axis=1).astype(jnp.float32)


def _head_kernel(padd_ref, pmax_ref, ci_ref, wf1_ref, bf1_ref,
                 wf2_ref, bf2_ref, out_ref):
    addt = jnp.sum(padd_ref[...], axis=0)
    maxt = jnp.max(pmax_ref[...], axis=0)
    ci = ci_ref[...]
    meant = addt * ci
    maxt = jnp.where(ci > 0.0, maxt, 0.0)
    cat = jnp.concatenate([meant, maxt, addt], axis=0)
    y = (jnp.dot(wf1_ref[...], cat.astype(jnp.bfloat16),
                 preferred_element_type=jnp.float32) + bf1_ref[...])
    y = _leaky(y)
    out_ref[...] = (jnp.dot(wf2_ref[...], y.astype(jnp.bfloat16),
                            preferred_element_type=jnp.float32)
                    + bf2_ref[...])


def kernel(x, edge_index, batch, w1l, w1r, b1, w2l, w2r, b2, w3l, w3r, b3,
           wu, bu, wf1, bf1, wf2, bf2):
    n = x.shape[0]
    num_graphs = G
    nj = n // 2 if n >= 512 else n
    nk1 = min(NK1, n)
    nkc = min(NKC, n)
    grid_j = n // nj

    src = edge_index[0]
    dst = edge_index[1]

    lin = src * n + dst
    e2 = lin.shape[0] // 2
    adjf = jnp.zeros((n * n,), jnp.float32).at[lin[:e2]].add(
        1.0, indices_are_sorted=True, mode="promise_in_bounds")
    adjf2 = jnp.zeros((n * n,), jnp.float32).at[lin[e2:]].add(
        1.0, indices_are_sorted=True, mode="promise_in_bounds")

    xrow_f32 = x.reshape(1, n)
    xwlt = (jnp.transpose(w1l) * xrow_f32).astype(jnp.bfloat16)
    xs = jnp.concatenate(
        [xwlt, jnp.ones((1, n), jnp.bfloat16), jnp.zeros((7, n), jnp.bfloat16)],
        axis=0)

    batch_row = batch.astype(jnp.float32).reshape(1, n)
    poolt = (batch[:, None] == jnp.arange(num_graphs, dtype=batch.dtype)
             [None, :]).astype(jnp.bfloat16)
    cnt = jnp.sum(poolt.astype(jnp.float32), axis=0).reshape(1, num_graphs)
    ci_row = jnp.where(cnt > 0, 1.0 / cnt, 0.0)

    w1rc = w1r.reshape(H, 1)
    b1c = b1.reshape(H, 1)
    w2lt = jnp.transpose(w2l).astype(jnp.bfloat16)
    w2rt = jnp.transpose(w2r).astype(jnp.bfloat16)
    b2c = b2.reshape(H, 1)
    w3lt = jnp.transpose(w3l).astype(jnp.bfloat16)
    w3rt = jnp.transpose(w3r).astype(jnp.bfloat16)
    b3c = b3.reshape(H, 1)
    wut = jnp.transpose(wu).astype(jnp.bfloat16)
    buc = bu.reshape(F_UP, 1)
    wf1t = jnp.transpose(wf1).astype(jnp.bfloat16)
    bf1c = bf1.reshape(H, 1)
    wf2t8 = jnp.zeros((8, H), jnp.float32).at[0, :].set(
        wf2[:, 0]).astype(jnp.bfloat16)

    bspec16 = pl.BlockSpec((nkc, nj), lambda j, k: (k, j))
    full2 = lambda shape: pl.BlockSpec(shape, lambda j, k: (0, 0))
    colblk = lambda rows: pl.BlockSpec((rows, nj), lambda j, k: (0, j))

    conv_params = pltpu.CompilerParams(
        dimension_semantics=("parallel", "arbitrary"),
        vmem_limit_bytes=100 << 20)

    ncore = 2 if n >= 512 else 1
    ksteps_core = n // nk1 // ncore
    parts, adjb = pl.pallas_call(
        functools.partial(_pass1_kernel, ksteps_core=ksteps_core, nk=nk1, n=n),
        grid=(ncore, ksteps_core),
        in_specs=[pl.BlockSpec((nk1 * n,),
                               lambda c, k: (c * ksteps_core + k,)),
                  pl.BlockSpec((nk1 * n,),
                               lambda c, k: (c * ksteps_core + k,)),
                  full2((40, n))],
        out_specs=[pl.BlockSpec((1, 40, n), lambda c, k: (c, 0, 0)),
                   pl.BlockSpec((nk1, n), lambda c, k: (c * ksteps_core + k, 0))],
        out_shape=[jax.ShapeDtypeStruct((ncore, 40, n), jnp.float32),
                   jax.ShapeDtypeStruct((n, n), jnp.bfloat16)],
        scratch_shapes=[pltpu.VMEM((40, n), jnp.float32)],
        compiler_params=conv_params,
        cost_estimate=pl.CostEstimate(
            flops=int(2 * 40 * n * n), transcendentals=0,
            bytes_accessed=int(n * n * 6)),
    )(adjf, adjf2, xs)

    h1t, dinv = pl.pallas_call(
        _combine1_kernel,
        out_shape=[jax.ShapeDtypeStruct((H, n), jnp.bfloat16),
                   jax.ShapeDtypeStruct((1, n), jnp.float32)],
    )(parts, xrow_f32, w1rc, b1c)


    h2t = pl.pallas_call(
        functools.partial(_conv2_kernel, nk_steps=n // nkc, nk=nkc, nj=nj),
        grid=(grid_j, n // nkc),
        in_specs=[bspec16, full2((H, n)),
                  full2((H, H)), full2((H, H)), full2((H, 1))],
        out_specs=colblk(H),
        out_shape=jax.ShapeDtypeStruct((H, n), jnp.bfloat16),
        scratch_shapes=[pltpu.VMEM((H, nj), jnp.float32)],
        compiler_params=conv_params,
        cost_estimate=pl.CostEstimate(
            flops=int(2 * H * n * n), transcendentals=0,
            bytes_accessed=int(n * n * 2)),
    )(adjb, h1t, w2lt, w2rt, b2c)

    padd, pmax = pl.pallas_call(
        functools.partial(_conv3_pool_kernel, nk_steps=n // nkc, nk=nkc, nj=nj),
        grid=(grid_j, n // nkc),
        in_specs=[bspec16, full2((H, n)), colblk(1), colblk(1),
                  pl.BlockSpec((nj, num_graphs), lambda j, k: (j, 0)),
                  full2((H, H)), full2((H, H)), full2((H, 1)),
                  full2((F_UP, H)), full2((F_UP, 1))],
        out_specs=[
            pl.BlockSpec((1, F_UP, num_graphs), lambda j, k: (j, 0, 0)),
            pl.BlockSpec((1, F_UP, num_graphs), lambda j, k: (j, 0, 0))],
        out_shape=[
            jax.ShapeDtypeStruct((grid_j, F_UP, num_graphs), jnp.float32),
            jax.ShapeDtypeStruct((grid_j, F_UP, num_graphs), jnp.float32)],
        scratch_shapes=[pltpu.VMEM((H, nj), jnp.float32)],
        compiler_params=conv_params,
        cost_estimate=pl.CostEstimate(
            flops=int(2 * H * n * n), transcendentals=0,
            bytes_accessed=int(n * n * 2)),
    )(adjb, h2t, dinv, batch_row, poolt,
      w3lt, w3rt, b3c, wut, buc)

    outt = pl.pallas_call(
        _head_kernel,
        out_shape=jax.ShapeDtypeStruct((8, num_graphs), jnp.float32),
    )(padd, pmax, ci_row, wf1t, bf1c, wf2t8, bf2)

    return jnp.transpose(outt[0:1, :num_graphs])


# speedup vs baseline: 1.6119x; 1.6119x over previous
import functools

import jax
import jax.numpy as jnp
from jax.experimental import pallas as pl
from jax.experimental.pallas import tpu as pltpu

NEG_SLOPE = 0.01
H = 32
F_UP = 128
G = 64
NK1 = 256
NKC = 512


def _leaky(x):
    return jnp.where(x > 0, x, NEG_SLOPE * x)


def _pass1_kernel(b_ref, xs_ref, part_ref, bb_ref, acc, *,
                  ksteps_core, nk, n):
    c = pl.program_id(0)
    k = pl.program_id(1)

    @pl.when(k == 0)
    def _():
        acc[...] = jnp.zeros_like(acc)

    bb = b_ref[...].reshape(nk, n).astype(jnp.bfloat16)
    bb_ref[...] = bb
    kg = c * ksteps_core + k
    lhs = xs_ref[:, pl.ds(pl.multiple_of(kg * nk, nk), nk)]
    acc[...] += jnp.dot(lhs, bb, preferred_element_type=jnp.float32)

    @pl.when(k == ksteps_core - 1)
    def _():
        part_ref[0, :, :] = acc[...]


def _combine1_kernel(part_ref, xrow_ref, w1r_ref, b1_ref, h1_ref, dinv_ref):
    acc = jnp.sum(part_ref[...], axis=0)
    agg1 = acc[0:H, :]
    deg = acc[H:H + 1, :]
    dinv_ref[...] = jnp.where(deg > 0, 1.0 / deg, 0.0)
    h = agg1 + (w1r_ref[...] * xrow_ref[...]) + b1_ref[...]
    h1_ref[...] = _leaky(h).astype(jnp.bfloat16)


def _conv2_kernel(b_ref, ht_ref, wl_ref, wr_ref, bias_ref,
                  out_ref, acc, *, nk_steps, nk, nj):
    j = pl.program_id(0)
    k = pl.program_id(1)

    @pl.when(k == 0)
    def _():
        acc[...] = jnp.zeros_like(acc)

    lhs = ht_ref[:, pl.ds(pl.multiple_of(k * nk, nk), nk)]
    acc[...] += jnp.dot(lhs, b_ref[...], preferred_element_type=jnp.float32)

    @pl.when(k == nk_steps - 1)
    def _():
        root = ht_ref[:, pl.ds(pl.multiple_of(j * nj, nj), nj)]
        y = (jnp.dot(wl_ref[...], acc[...].astype(jnp.bfloat16),
                     preferred_element_type=jnp.float32)
             + jnp.dot(wr_ref[...], root,
                       preferred_element_type=jnp.float32)
             + bias_ref[...])
        out_ref[...] = _leaky(y).astype(jnp.bfloat16)


def _conv3_pool_kernel(b_ref, ht_ref, dinv_ref, batch_ref, pool_ref,
                       wl_ref, wr_ref, bias_ref, wu_ref, bu_ref,
                       padd_ref, pmax_ref, acc, *, nk_steps, nk, nj):
    j = pl.program_id(0)
    k = pl.program_id(1)

    @pl.when(k == 0)
    def _():
        acc[...] = jnp.zeros_like(acc)

    lhs = ht_ref[:, pl.ds(pl.multiple_of(k * nk, nk), nk)]
    acc[...] += jnp.dot(lhs, b_ref[...], preferred_element_type=jnp.float32)

    @pl.when(k == nk_steps - 1)
    def _():
        aggm = acc[...] * dinv_ref[...]
        root = ht_ref[:, pl.ds(pl.multiple_of(j * nj, nj), nj)]
        y = (jnp.dot(wl_ref[...], aggm.astype(jnp.bfloat16),
                     preferred_element_type=jnp.float32)
             + jnp.dot(wr_ref[...], root,
                       preferred_element_type=jnp.float32)
             + bias_ref[...])
        y = _leaky(y)
        z = jnp.dot(wu_ref[...], y.astype(jnp.bfloat16),
                    preferred_element_type=jnp.float32) + bu_ref[...]
        z = _leaky(z)
        zb = z.astype(jnp.bfloat16)
        padd_ref[0, :, :] = jnp.dot(zb, pool_ref[...],
                              preferred_element_type=jnp.float32)
        brow = batch_ref[...]
        neg = jnp.bfloat16(-jnp.inf)
        maxes = []
        for g in range(G):
            masked = jnp.where(brow == jnp.float32(g), zb, neg)
            maxes.append(jnp.max(masked, axis=1))
        pmax_ref[0, :, :] = jnp.stack(maxes, axis=1).astype(jnp.float32)


def _head_kernel(padd_ref, pmax_ref, ci_ref, wf1_ref, bf1_ref,
                 wf2_ref, bf2_ref, out_ref):
    addt = jnp.sum(padd_ref[...], axis=0)
    maxt = jnp.max(pmax_ref[...], axis=0)
    ci = ci_ref[...]
    meant = addt * ci
    maxt = jnp.where(ci > 0.0, maxt, 0.0)
    cat = jnp.concatenate([meant, maxt, addt], axis=0)
    y = (jnp.dot(wf1_ref[...], cat.astype(jnp.bfloat16),
                 preferred_element_type=jnp.float32) + bf1_ref[...])
    y = _leaky(y)
    out_ref[...] = (jnp.dot(wf2_ref[...], y.astype(jnp.bfloat16),
                            preferred_element_type=jnp.float32)
                    + bf2_ref[...])


def kernel(x, edge_index, batch, w1l, w1r, b1, w2l, w2r, b2, w3l, w3r, b3,
           wu, bu, wf1, bf1, wf2, bf2):
    n = x.shape[0]
    num_graphs = G
    nj = n // 2 if n >= 512 else n
    nk1 = min(NK1, n)
    nkc = min(NKC, n)
    grid_j = n // nj

    src = edge_index[0]
    dst = edge_index[1]

    lin = src * n + dst
    adjf = jnp.zeros((n * n,), jnp.float32).at[lin].add(
        1.0, indices_are_sorted=True, mode="promise_in_bounds")

    xrow_f32 = x.reshape(1, n)
    xwlt = (jnp.transpose(w1l) * xrow_f32).astype(jnp.bfloat16)
    xs = jnp.concatenate(
        [xwlt, jnp.ones((1, n), jnp.bfloat16), jnp.zeros((7, n), jnp.bfloat16)],
        axis=0)

    batch_row = batch.astype(jnp.float32).reshape(1, n)
    poolt = (batch[:, None] == jnp.arange(num_graphs, dtype=batch.dtype)
             [None, :]).astype(jnp.bfloat16)
    cnt = jnp.sum(poolt.astype(jnp.float32), axis=0).reshape(1, num_graphs)
    ci_row = jnp.where(cnt > 0, 1.0 / cnt, 0.0)

    w1rc = w1r.reshape(H, 1)
    b1c = b1.reshape(H, 1)
    w2lt = jnp.transpose(w2l).astype(jnp.bfloat16)
    w2rt = jnp.transpose(w2r).astype(jnp.bfloat16)
    b2c = b2.reshape(H, 1)
    w3lt = jnp.transpose(w3l).astype(jnp.bfloat16)
    w3rt = jnp.transpose(w3r).astype(jnp.bfloat16)
    b3c = b3.reshape(H, 1)
    wut = jnp.transpose(wu).astype(jnp.bfloat16)
    buc = bu.reshape(F_UP, 1)
    wf1t = jnp.transpose(wf1).astype(jnp.bfloat16)
    bf1c = bf1.reshape(H, 1)
    wf2t8 = jnp.zeros((8, H), jnp.float32).at[0, :].set(
        wf2[:, 0]).astype(jnp.bfloat16)

    bspec16 = pl.BlockSpec((nkc, nj), lambda j, k: (k, j))
    full2 = lambda shape: pl.BlockSpec(shape, lambda j, k: (0, 0))
    colblk = lambda rows: pl.BlockSpec((rows, nj), lambda j, k: (0, j))

    conv_params = pltpu.CompilerParams(
        dimension_semantics=("parallel", "arbitrary"),
        vmem_limit_bytes=100 << 20)

    ncore = 2 if n >= 512 else 1
    ksteps_core = n // nk1 // ncore
    parts, adjb = pl.pallas_call(
        functools.partial(_pass1_kernel, ksteps_core=ksteps_core, nk=nk1, n=n),
        grid=(ncore, ksteps_core),
        in_specs=[pl.BlockSpec((nk1 * n,),
                               lambda c, k: (c * ksteps_core + k,)),
                  full2((40, n))],
        out_specs=[pl.BlockSpec((1, 40, n), lambda c, k: (c, 0, 0)),
                   pl.BlockSpec((nk1, n), lambda c, k: (c * ksteps_core + k, 0))],
        out_shape=[jax.ShapeDtypeStruct((ncore, 40, n), jnp.float32),
                   jax.ShapeDtypeStruct((n, n), jnp.bfloat16)],
        scratch_shapes=[pltpu.VMEM((40, n), jnp.float32)],
        compiler_params=conv_params,
        cost_estimate=pl.CostEstimate(
            flops=int(2 * 40 * n * n), transcendentals=0,
            bytes_accessed=int(n * n * 6)),
    )(adjf, xs)

    h1t, dinv = pl.pallas_call(
        _combine1_kernel,
        out_shape=[jax.ShapeDtypeStruct((H, n), jnp.bfloat16),
                   jax.ShapeDtypeStruct((1, n), jnp.float32)],
    )(parts, xrow_f32, w1rc, b1c)


    h2t = pl.pallas_call(
        functools.partial(_conv2_kernel, nk_steps=n // nkc, nk=nkc, nj=nj),
        grid=(grid_j, n // nkc),
        in_specs=[bspec16, full2((H, n)),
                  full2((H, H)), full2((H, H)), full2((H, 1))],
        out_specs=colblk(H),
        out_shape=jax.ShapeDtypeStruct((H, n), jnp.bfloat16),
        scratch_shapes=[pltpu.VMEM((H, nj), jnp.float32)],
        compiler_params=conv_params,
        cost_estimate=pl.CostEstimate(
            flops=int(2 * H * n * n), transcendentals=0,
            bytes_accessed=int(n * n * 2)),
    )(adjb, h1t, w2lt, w2rt, b2c)

    padd, pmax = pl.pallas_call(
        functools.partial(_conv3_pool_kernel, nk_steps=n // nkc, nk=nkc, nj=nj),
        grid=(grid_j, n // nkc),
        in_specs=[bspec16, full2((H, n)), colblk(1), colblk(1),
                  pl.BlockSpec((nj, num_graphs), lambda j, k: (j, 0)),
                  full2((H, H)), full2((H, H)), full2((H, 1)),
                  full2((F_UP, H)), full2((F_UP, 1))],
        out_specs=[
            pl.BlockSpec((1, F_UP, num_graphs), lambda j, k: (j, 0, 0)),
            pl.BlockSpec((1, F_UP, num_graphs), lambda j, k: (j, 0, 0))],
        out_shape=[
            jax.ShapeDtypeStruct((grid_j, F_UP, num_graphs), jnp.float32),
            jax.ShapeDtypeStruct((grid_j, F_UP, num_graphs), jnp.float32)],
        scratch_shapes=[pltpu.VMEM((H, nj), jnp.float32)],
        compiler_params=conv_params,
        cost_estimate=pl.CostEstimate(
            flops=int(2 * H * n * n), transcendentals=0,
            bytes_accessed=int(n * n * 2)),
    )(adjb, h2t, dinv, batch_row, poolt,
      w3lt, w3rt, b3c, wut, buc)

    outt = pl.pallas_call(
        _head_kernel,
        out_shape=jax.ShapeDtypeStruct((8, num_graphs), jnp.float32),
    )(padd, pmax, ci_row, wf1t, bf1c, wf2t8, bf2)

    return jnp.transpose(outt[0:1, :num_graphs])


# speedup vs baseline: 1.6142x; 1.0015x over previous
import functools

import jax
import jax.numpy as jnp
from jax.experimental import pallas as pl
from jax.experimental.pallas import tpu as pltpu

NEG_SLOPE = 0.01
H = 32
F_UP = 128
G = 64
NK1 = 256
NKC = 1024


def _leaky(x):
    return jnp.where(x > 0, x, NEG_SLOPE * x)


def _pass1_kernel(b_ref, xs_ref, part_ref, bb_ref, acc, *,
                  ksteps_core, nk, n):
    c = pl.program_id(0)
    k = pl.program_id(1)

    @pl.when(k == 0)
    def _():
        acc[...] = jnp.zeros_like(acc)

    bb = b_ref[...].reshape(nk, n).astype(jnp.bfloat16)
    bb_ref[...] = bb
    kg = c * ksteps_core + k
    lhs = xs_ref[:, pl.ds(pl.multiple_of(kg * nk, nk), nk)]
    acc[...] += jnp.dot(lhs, bb, preferred_element_type=jnp.float32)

    @pl.when(k == ksteps_core - 1)
    def _():
        part_ref[0, :, :] = acc[...]


def _combine1_kernel(part_ref, xrow_ref, w1r_ref, b1_ref, h1_ref, dinv_ref):
    acc = jnp.sum(part_ref[...], axis=0)
    agg1 = acc[0:H, :]
    deg = acc[H:H + 1, :]
    dinv_ref[...] = jnp.where(deg > 0, 1.0 / deg, 0.0)
    h = agg1 + (w1r_ref[...] * xrow_ref[...]) + b1_ref[...]
    h1_ref[...] = _leaky(h).astype(jnp.bfloat16)


def _conv2_kernel(b_ref, ht_ref, wl_ref, wr_ref, bias_ref,
                  out_ref, acc, *, nk_steps, nk, nj):
    j = pl.program_id(0)
    k = pl.program_id(1)

    @pl.when(k == 0)
    def _():
        acc[...] = jnp.zeros_like(acc)

    lhs = ht_ref[:, pl.ds(pl.multiple_of(k * nk, nk), nk)]
    acc[...] += jnp.dot(lhs, b_ref[...], preferred_element_type=jnp.float32)

    @pl.when(k == nk_steps - 1)
    def _():
        root = ht_ref[:, pl.ds(pl.multiple_of(j * nj, nj), nj)]
        y = (jnp.dot(wl_ref[...], acc[...].astype(jnp.bfloat16),
                     preferred_element_type=jnp.float32)
             + jnp.dot(wr_ref[...], root,
                       preferred_element_type=jnp.float32)
             + bias_ref[...])
        out_ref[...] = _leaky(y).astype(jnp.bfloat16)


def _conv3_pool_kernel(b_ref, ht_ref, dinv_ref, batch_ref, pool_ref,
                       wl_ref, wr_ref, bias_ref, wu_ref, bu_ref,
                       padd_ref, pmax_ref, acc, *, nk_steps, nk, nj):
    j = pl.program_id(0)
    k = pl.program_id(1)

    @pl.when(k == 0)
    def _():
        acc[...] = jnp.zeros_like(acc)

    lhs = ht_ref[:, pl.ds(pl.multiple_of(k * nk, nk), nk)]
    acc[...] += jnp.dot(lhs, b_ref[...], preferred_element_type=jnp.float32)

    @pl.when(k == nk_steps - 1)
    def _():
        aggm = acc[...] * dinv_ref[...]
        root = ht_ref[:, pl.ds(pl.multiple_of(j * nj, nj), nj)]
        y = (jnp.dot(wl_ref[...], aggm.astype(jnp.bfloat16),
                     preferred_element_type=jnp.float32)
             + jnp.dot(wr_ref[...], root,
                       preferred_element_type=jnp.float32)
             + bias_ref[...])
        y = _leaky(y)
        z = jnp.dot(wu_ref[...], y.astype(jnp.bfloat16),
                    preferred_element_type=jnp.float32) + bu_ref[...]
        z = _leaky(z)
        zb = z.astype(jnp.bfloat16)
        padd_ref[0, :, :] = jnp.dot(zb, pool_ref[...],
                              preferred_element_type=jnp.float32)
        brow = batch_ref[...]
        neg = jnp.bfloat16(-jnp.inf)
        maxes = []
        for g in range(G):
            masked = jnp.where(brow == jnp.float32(g), zb, neg)
            maxes.append(jnp.max(masked, axis=1))
        pmax_ref[0, :, :] = jnp.stack(maxes, axis=1).astype(jnp.float32)


def _head_kernel(padd_ref, pmax_ref, ci_ref, wf1_ref, bf1_ref,
                 wf2_ref, bf2_ref, out_ref):
    addt = jnp.sum(padd_ref[...], axis=0)
    maxt = jnp.max(pmax_ref[...], axis=0)
    ci = ci_ref[...]
    meant = addt * ci
    maxt = jnp.where(ci > 0.0, maxt, 0.0)
    cat = jnp.concatenate([meant, maxt, addt], axis=0)
    y = (jnp.dot(wf1_ref[...], cat.astype(jnp.bfloat16),
                 preferred_element_type=jnp.float32) + bf1_ref[...])
    y = _leaky(y)
    out_ref[...] = (jnp.dot(wf2_ref[...], y.astype(jnp.bfloat16),
                            preferred_element_type=jnp.float32)
                    + bf2_ref[...])


def kernel(x, edge_index, batch, w1l, w1r, b1, w2l, w2r, b2, w3l, w3r, b3,
           wu, bu, wf1, bf1, wf2, bf2):
    n = x.shape[0]
    num_graphs = G
    nj = n // 2 if n >= 512 else n
    nk1 = min(NK1, n)
    nkc = min(NKC, n)
    grid_j = n // nj

    src = edge_index[0]
    dst = edge_index[1]

    lin = src * n + dst
    adjf = jnp.zeros((n * n,), jnp.float32).at[lin].add(
        1.0, indices_are_sorted=True, mode="promise_in_bounds")

    xrow_f32 = x.reshape(1, n)
    xwlt = (jnp.transpose(w1l) * xrow_f32).astype(jnp.bfloat16)
    xs = jnp.concatenate(
        [xwlt, jnp.ones((1, n), jnp.bfloat16), jnp.zeros((7, n), jnp.bfloat16)],
        axis=0)

    batch_row = batch.astype(jnp.float32).reshape(1, n)
    poolt = (batch[:, None] == jnp.arange(num_graphs, dtype=batch.dtype)
             [None, :]).astype(jnp.bfloat16)
    cnt = jnp.sum(poolt.astype(jnp.float32), axis=0).reshape(1, num_graphs)
    ci_row = jnp.where(cnt > 0, 1.0 / cnt, 0.0)

    w1rc = w1r.reshape(H, 1)
    b1c = b1.reshape(H, 1)
    w2lt = jnp.transpose(w2l).astype(jnp.bfloat16)
    w2rt = jnp.transpose(w2r).astype(jnp.bfloat16)
    b2c = b2.reshape(H, 1)
    w3lt = jnp.transpose(w3l).astype(jnp.bfloat16)
    w3rt = jnp.transpose(w3r).astype(jnp.bfloat16)
    b3c = b3.reshape(H, 1)
    wut = jnp.transpose(wu).astype(jnp.bfloat16)
    buc = bu.reshape(F_UP, 1)
    wf1t = jnp.transpose(wf1).astype(jnp.bfloat16)
    bf1c = bf1.reshape(H, 1)
    wf2t8 = jnp.zeros((8, H), jnp.float32).at[0, :].set(
        wf2[:, 0]).astype(jnp.bfloat16)

    bspec16 = pl.BlockSpec((nkc, nj), lambda j, k: (k, j))
    full2 = lambda shape: pl.BlockSpec(shape, lambda j, k: (0, 0))
    colblk = lambda rows: pl.BlockSpec((rows, nj), lambda j, k: (0, j))

    conv_params = pltpu.CompilerParams(
        dimension_semantics=("parallel", "arbitrary"),
        vmem_limit_bytes=100 << 20)

    ncore = 2 if n >= 512 else 1
    ksteps_core = n // nk1 // ncore
    parts, adjb = pl.pallas_call(
        functools.partial(_pass1_kernel, ksteps_core=ksteps_core, nk=nk1, n=n),
        grid=(ncore, ksteps_core),
        in_specs=[pl.BlockSpec((nk1 * n,),
                               lambda c, k: (c * ksteps_core + k,)),
                  full2((40, n))],
        out_specs=[pl.BlockSpec((1, 40, n), lambda c, k: (c, 0, 0)),
                   pl.BlockSpec((nk1, n), lambda c, k: (c * ksteps_core + k, 0))],
        out_shape=[jax.ShapeDtypeStruct((ncore, 40, n), jnp.float32),
                   jax.ShapeDtypeStruct((n, n), jnp.bfloat16)],
        scratch_shapes=[pltpu.VMEM((40, n), jnp.float32)],
        compiler_params=conv_params,
        cost_estimate=pl.CostEstimate(
            flops=int(2 * 40 * n * n), transcendentals=0,
            bytes_accessed=int(n * n * 6)),
    )(adjf, xs)

    h1t, dinv = pl.pallas_call(
        _combine1_kernel,
        out_shape=[jax.ShapeDtypeStruct((H, n), jnp.bfloat16),
                   jax.ShapeDtypeStruct((1, n), jnp.float32)],
    )(parts, xrow_f32, w1rc, b1c)


    h2t = pl.pallas_call(
        functools.partial(_conv2_kernel, nk_steps=n // nkc, nk=nkc, nj=nj),
        grid=(grid_j, n // nkc),
        in_specs=[bspec16, full2((H, n)),
                  full2((H, H)), full2((H, H)), full2((H, 1))],
        out_specs=colblk(H),
        out_shape=jax.ShapeDtypeStruct((H, n), jnp.bfloat16),
        scratch_shapes=[pltpu.VMEM((H, nj), jnp.float32)],
        compiler_params=conv_params,
        cost_estimate=pl.CostEstimate(
            flops=int(2 * H * n * n), transcendentals=0,
            bytes_accessed=int(n * n * 2)),
    )(adjb, h1t, w2lt, w2rt, b2c)

    padd, pmax = pl.pallas_call(
        functools.partial(_conv3_pool_kernel, nk_steps=n // nkc, nk=nkc, nj=nj),
        grid=(grid_j, n // nkc),
        in_specs=[bspec16, full2((H, n)), colblk(1), colblk(1),
                  pl.BlockSpec((nj, num_graphs), lambda j, k: (j, 0)),
                  full2((H, H)), full2((H, H)), full2((H, 1)),
                  full2((F_UP, H)), full2((F_UP, 1))],
        out_specs=[
            pl.BlockSpec((1, F_UP, num_graphs), lambda j, k: (j, 0, 0)),
            pl.BlockSpec((1, F_UP, num_graphs), lambda j, k: (j, 0, 0))],
        out_shape=[
            jax.ShapeDtypeStruct((grid_j, F_UP, num_graphs), jnp.float32),
            jax.ShapeDtypeStruct((grid_j, F_UP, num_graphs), jnp.float32)],
        scratch_shapes=[pltpu.VMEM((H, nj), jnp.float32)],
        compiler_params=conv_params,
        cost_estimate=pl.CostEstimate(
            flops=int(2 * H * n * n), transcendentals=0,
            bytes_accessed=int(n * n * 2)),
    )(adjb, h2t, dinv, batch_row, poolt,
      w3lt, w3rt, b3c, wut, buc)

    outt = pl.pallas_call(
        _head_kernel,
        out_shape=jax.ShapeDtypeStruct((8, num_graphs), jnp.float32),
    )(padd, pmax, ci_row, wf1t, bf1c, wf2t8, bf2)

    return jnp.transpose(outt[0:1, :num_graphs])
